# baseline (speedup 1.0000x reference)
import jax
import jax.numpy as jnp
from jax.experimental import pallas as pl

_K = 64
_ROWS = 128
_COLS = 32768
_BLK = 8


def _tc_body(x_ref, o_ref):
    v = x_ref[...]
    col = jax.lax.broadcasted_iota(jnp.int32, v.shape, 1)
    kiota = jax.lax.broadcasted_iota(jnp.int32, (_BLK, _K), 1)
    big = jnp.int32(2**30)

    def body(k, carry):
        v, acc = carry
        m = jnp.max(v, axis=1, keepdims=True)
        cand = jnp.where(v == m, col, big)
        idx = jnp.min(cand, axis=1, keepdims=True)
        acc = jnp.where(kiota == k, idx, acc)
        v = jnp.where(cand == idx, -jnp.inf, v)
        return v, acc

    _, acc = jax.lax.fori_loop(0, _K, body, (v, jnp.zeros((_BLK, _K), jnp.int32)))
    o_ref[...] = acc


def kernel(x):
    return pl.pallas_call(
        _tc_body,
        grid=(_ROWS // _BLK,),
        in_specs=[pl.BlockSpec((_BLK, _COLS), lambda i: (i, 0))],
        out_specs=pl.BlockSpec((_BLK, _K), lambda i: (i, 0)),
        out_shape=jax.ShapeDtypeStruct((_ROWS, _K), jnp.int32),
    )(x)


# speedup vs baseline: 3.1313x; 3.1313x over previous
import functools

import jax
import jax.numpy as jnp
from jax import lax
from jax.experimental import pallas as pl
from jax.experimental.pallas import tpu as pltpu
from jax.experimental.pallas import tpu_sc as plsc

_K = 64
_ROWS = 128
_COLS = 32768
_NC = 2
_NS = 16
_NW = _NC * _NS
_RPW = _ROWS // _NW
_NV = _COLS // 16
_CAND = _COLS + 16

_BIG = 2**30


def _lane():
    return lax.broadcasted_iota(jnp.int32, (16,), 0)


def _skey(v):
    b = lax.bitcast_convert_type(v, jnp.int32)
    return b ^ ((b >> 31) & jnp.int32(0x7FFFFFFF))


def _clear_hist(hist):
    z = jnp.zeros((16,), jnp.int32)

    def b(i, c):
        hist[pl.ds(pl.multiple_of(i * 16, 16), 16)] = z
        return c

    lax.fori_loop(0, 256, b, 0)


def _scan_bins(hist, sfx, need):
    lane = _lane()

    def chunk(t, r):
        c = 15 - t
        idx0 = c * 256 + lane * 16

        def gsum(l, acc):
            return acc + plsc.load_gather(hist, [idx0 + l])

        acc = lax.fori_loop(0, 16, gsum, jnp.zeros((16,), jnp.int32))
        cs = plsc.cumsum(lax.rev(acc, (0,)))
        sfx[pl.ds(pl.multiple_of(c * 16, 16), 16)] = lax.rev(cs, (0,)) + r
        return r + jnp.max(cs)

    lax.fori_loop(0, 16, chunk, jnp.int32(0))

    def cnt(c, acc):
        s = sfx[pl.ds(pl.multiple_of(c * 16, 16), 16)]
        return acc + (s >= need).astype(jnp.int32)

    accv = lax.fori_loop(0, 16, cnt, jnp.zeros((16,), jnp.int32))
    bbin = jnp.sum(accv) - 1
    g = plsc.load_gather(sfx, [jnp.broadcast_to(jnp.minimum(bbin + 1, 255), (16,))])
    above = jnp.where(bbin >= 255, jnp.int32(0), jnp.max(g))
    return bbin, above


def _filter(row_v, cand, defb, cand_n, def_n, bbin, sh, flip):
    lane = _lane()

    def fbody(i, carry):
        doff, coff = carry
        lm = (i * 16 + lane) < cand_n
        ci = cand[pl.ds(pl.multiple_of(i * 16, 16), 16)] & 0x7FFF
        s = _skey(plsc.load_gather(row_v, [ci]))
        byte = lax.shift_right_logical(s, sh) & 0xFF
        if flip:
            byte = byte ^ 0x80
        dm = (byte > bbin) & lm
        bm = (byte == bbin) & lm
        dmi = dm.astype(jnp.int32)
        bmi = bm.astype(jnp.int32)
        plsc.store_scatter(defb, [plsc.cumsum(dmi) - dmi + doff], ci, mask=dm)
        plsc.store_scatter(cand, [plsc.cumsum(bmi) - bmi + coff], ci, mask=bm)
        return (doff + plsc.all_reduce_population_count(dm),
                coff + plsc.all_reduce_population_count(bm))

    doff, coff = lax.fori_loop(
        0, (cand_n + 15) // 16, fbody,
        (jnp.broadcast_to(def_n, (16,)), jnp.zeros((16,), jnp.int32)))
    return jnp.max(doff), jnp.max(coff)


def _refine(row_v, hist, sfx, cand, defb, def_n, cand_n, sh):
    lane = _lane()
    ones = jnp.ones((16,), jnp.int32)

    def do(args):
        def_n, cand_n = args
        _clear_hist(hist)

        def hb(i, c):
            lm = (i * 16 + lane) < cand_n
            ci = cand[pl.ds(pl.multiple_of(i * 16, 16), 16)] & 0x7FFF
            s = _skey(plsc.load_gather(row_v, [ci]))
            byte = lax.shift_right_logical(s, sh) & 0xFF
            plsc.addupdate_scatter(hist, [(byte << 4) + lane], ones, mask=lm)
            return c

        lax.fori_loop(0, (cand_n + 15) // 16, hb, 0)
        bbin, _ = _scan_bins(hist, sfx, _K - def_n)
        return _filter(row_v, cand, defb, cand_n, def_n, bbin, sh, False)

    return lax.cond(cand_n > _K - def_n, do, lambda a: a, (def_n, cand_n))


def _final_sort(row_v, defb, outv):
    lane = _lane()
    iv = [defb[pl.ds(16 * j, 16)] for j in range(4)]
    vv = [plsc.load_gather(row_v, [iv[j] & 0x7FFF]) for j in range(4)]
    ninf = jnp.float32(-jnp.inf)

    def kb(k, carry):
        v0, v1, v2, v3, acc = carry
        ms = jnp.max(jnp.maximum(jnp.maximum(v0, v1), jnp.maximum(v2, v3)))
        c0 = jnp.where(v0 == ms, iv[0], _BIG)
        c1 = jnp.where(v1 == ms, iv[1], _BIG)
        c2 = jnp.where(v2 == ms, iv[2], _BIG)
        c3 = jnp.where(v3 == ms, iv[3], _BIG)
        mi = jnp.min(jnp.minimum(jnp.minimum(c0, c1), jnp.minimum(c2, c3)))
        acc = jnp.where(lane == (k & 15), mi, acc)

        @pl.when((k & 15) == 15)
        def _():
            outv[pl.ds(pl.multiple_of(k - 15, 16), 16)] = acc

        v0 = jnp.where((v0 == ms) & (iv[0] == mi), ninf, v0)
        v1 = jnp.where((v1 == ms) & (iv[1] == mi), ninf, v1)
        v2 = jnp.where((v2 == ms) & (iv[2] == mi), ninf, v2)
        v3 = jnp.where((v3 == ms) & (iv[3] == mi), ninf, v3)
        return (v0, v1, v2, v3, acc)

    lax.fori_loop(0, _K, kb, (*vv, jnp.zeros((16,), jnp.int32)))


def _sc_body(x_hbm, out_hbm, row_v, hist, sfx, cand, defb, outv):
    wid = lax.axis_index("s") * _NC + lax.axis_index("c")
    lane = _lane()
    ones = jnp.ones((16,), jnp.int32)

    def row_body(j, carry):
        r = wid * _RPW + j
        pltpu.sync_copy(
            x_hbm.at[pl.ds(pl.multiple_of(r * _COLS, _COLS), _COLS)], row_v)

        _clear_hist(hist)

        def hbody(i, c):
            v = row_v[pl.ds(pl.multiple_of(i * 16, 16), 16)]
            s = _skey(v)
            addr = ((lax.shift_right_logical(s, 20) & 0xFF0) ^ 0x800) + lane
            plsc.addupdate_scatter(hist, [addr], ones)
            return c

        lax.fori_loop(0, _NV, hbody, 0)
        b1, _ = _scan_bins(hist, sfx, jnp.int32(_K))
        sbound = (b1 ^ 0x80) << 24

        def cbody(i, carry):
            off, base = carry
            s = _skey(row_v[pl.ds(pl.multiple_of(i * 16, 16), 16)])
            m = s >= sbound
            mi = m.astype(jnp.int32)
            plsc.store_scatter(cand, [plsc.cumsum(mi) - mi + off],
                               base + lane, mask=m)
            return (off + plsc.all_reduce_population_count(m), base + 16)

        off, _b = lax.fori_loop(
            0, _NV, cbody, (jnp.zeros((16,), jnp.int32),
                            jnp.zeros((16,), jnp.int32)))
        cand_n = jnp.max(off)

        def_n, cand_n = _filter(row_v, cand, defb, cand_n, jnp.int32(0),
                                b1, 24, True)
        for sh in (16, 8, 0):
            def_n, cand_n = _refine(row_v, hist, sfx, cand, defb,
                                    def_n, cand_n, sh)

        need_t = _K - def_n

        def abody(i, c):
            lm = (i * 16 + lane) < need_t
            ci = cand[pl.ds(pl.multiple_of(i * 16, 16), 16)] & 0x7FFF
            plsc.store_scatter(defb, [def_n + i * 16 + lane], ci, mask=lm)
            return c

        lax.fori_loop(0, (need_t + 15) // 16, abody, 0)

        _final_sort(row_v, defb, outv)
        pltpu.sync_copy(outv,
                        out_hbm.at[pl.ds(pl.multiple_of(r * _K, _K), _K)])
        return carry

    lax.fori_loop(0, _RPW, row_body, 0)


@functools.cache
def _sc_kernel():
    return pl.kernel(
        _sc_body,
        out_type=jax.ShapeDtypeStruct((_ROWS * _K,), jnp.int32),
        mesh=plsc.VectorSubcoreMesh(core_axis_name="c", subcore_axis_name="s",
                                    num_cores=_NC, num_subcores=_NS),
        scratch_types=[
            pltpu.VMEM((_COLS,), jnp.float32),
            pltpu.VMEM((4096,), jnp.int32),
            pltpu.VMEM((256,), jnp.int32),
            pltpu.VMEM((_CAND,), jnp.int32),
            pltpu.VMEM((96,), jnp.int32),
            pltpu.VMEM((_K,), jnp.int32),
        ],
        compiler_params=pltpu.CompilerParams(needs_layout_passes=False),
    )


def kernel(x):
    return _sc_kernel()(x.reshape(-1)).reshape(_ROWS, _K)


# speedup vs baseline: 8.7327x; 2.7889x over previous
import functools

import jax
import jax.numpy as jnp
from jax import lax
from jax.experimental import pallas as pl
from jax.experimental.pallas import tpu as pltpu
from jax.experimental.pallas import tpu_sc as plsc

_K = 64
_ROWS = 128
_COLS = 32768
_NC = 2
_NS = 16
_NW = _NC * _NS
_RPW = _ROWS // _NW
_NV = _COLS // 16
_CAND = _COLS + 16

_BIG = 2**30


def _lane():
    return lax.broadcasted_iota(jnp.int32, (16,), 0)


def _skey(v):
    b = lax.bitcast_convert_type(v, jnp.int32)
    return b ^ ((b >> 31) & jnp.int32(0x7FFFFFFF))


def _clear_hist(hist):
    z = jnp.zeros((16,), jnp.int32)

    @plsc.parallel_loop(0, 256, unroll=8)
    def _(i):
        hist[pl.ds(pl.multiple_of(i * 16, 16), 16)] = z


def _scan_bins(hist, sfx, need):
    lane = _lane()

    def chunk(t, r):
        c = 15 - t
        idx0 = c * 256 + lane * 16

        @plsc.parallel_loop(0, 16, unroll=4, carry=jnp.zeros((16,), jnp.int32))
        def acc(l, a):
            return a + plsc.load_gather(hist, [idx0 + l])
        cs = plsc.cumsum(lax.rev(acc, (0,)))
        sfx[pl.ds(pl.multiple_of(c * 16, 16), 16)] = lax.rev(cs, (0,)) + r
        return r + jnp.max(cs)

    lax.fori_loop(0, 16, chunk, jnp.int32(0))

    def cnt(c, acc):
        s = sfx[pl.ds(pl.multiple_of(c * 16, 16), 16)]
        return acc + (s >= need).astype(jnp.int32)

    accv = lax.fori_loop(0, 16, cnt, jnp.zeros((16,), jnp.int32))
    bbin = jnp.sum(accv) - 1
    g = plsc.load_gather(sfx, [jnp.broadcast_to(jnp.minimum(bbin + 1, 255), (16,))])
    above = jnp.where(bbin >= 255, jnp.int32(0), jnp.max(g))
    return bbin, above


def _filter(row_v, cand, defb, cand_n, def_n, bbin, sh, flip):
    lane = _lane()

    def fbody(i, carry):
        doff, coff = carry
        lm = (i * 16 + lane) < cand_n
        ci = cand[pl.ds(pl.multiple_of(i * 16, 16), 16)] & 0x7FFF
        s = _skey(plsc.load_gather(row_v, [ci]))
        byte = lax.shift_right_logical(s, sh) & 0xFF
        if flip:
            byte = byte ^ 0x80
        dm = (byte > bbin) & lm
        bm = (byte == bbin) & lm
        dmi = dm.astype(jnp.int32)
        bmi = bm.astype(jnp.int32)
        plsc.store_scatter(defb, [plsc.cumsum(dmi) - dmi + doff], ci, mask=dm)
        plsc.store_scatter(cand, [plsc.cumsum(bmi) - bmi + coff], ci, mask=bm)
        return (doff + plsc.all_reduce_population_count(dm),
                coff + plsc.all_reduce_population_count(bm))

    doff, coff = lax.fori_loop(
        0, (cand_n + 15) // 16, fbody,
        (jnp.broadcast_to(def_n, (16,)), jnp.zeros((16,), jnp.int32)))
    return jnp.max(doff), jnp.max(coff)


def _refine(row_v, hist, sfx, cand, defb, def_n, cand_n, sh):
    lane = _lane()
    ones = jnp.ones((16,), jnp.int32)

    def do(args):
        def_n, cand_n = args
        _clear_hist(hist)

        def hb(i, c):
            lm = (i * 16 + lane) < cand_n
            ci = cand[pl.ds(pl.multiple_of(i * 16, 16), 16)] & 0x7FFF
            s = _skey(plsc.load_gather(row_v, [ci]))
            byte = lax.shift_right_logical(s, sh) & 0xFF
            plsc.addupdate_scatter(hist, [(byte << 4) + lane], ones, mask=lm)
            return c

        lax.fori_loop(0, (cand_n + 15) // 16, hb, 0)
        bbin, _ = _scan_bins(hist, sfx, _K - def_n)
        return _filter(row_v, cand, defb, cand_n, def_n, bbin, sh, False)

    return lax.cond(cand_n > _K - def_n, do, lambda a: a, (def_n, cand_n))


def _final_sort(row_v, defb, outv):
    lane = _lane()
    iv = [defb[pl.ds(16 * j, 16)] for j in range(4)]
    vv = [plsc.load_gather(row_v, [iv[j] & 0x7FFF]) for j in range(4)]
    ninf = jnp.float32(-jnp.inf)

    def kb(k, carry):
        v0, v1, v2, v3, acc = carry
        ms = jnp.max(jnp.maximum(jnp.maximum(v0, v1), jnp.maximum(v2, v3)))
        c0 = jnp.where(v0 == ms, iv[0], _BIG)
        c1 = jnp.where(v1 == ms, iv[1], _BIG)
        c2 = jnp.where(v2 == ms, iv[2], _BIG)
        c3 = jnp.where(v3 == ms, iv[3], _BIG)
        mi = jnp.min(jnp.minimum(jnp.minimum(c0, c1), jnp.minimum(c2, c3)))
        acc = jnp.where(lane == (k & 15), mi, acc)

        @pl.when((k & 15) == 15)
        def _():
            outv[pl.ds(pl.multiple_of(k - 15, 16), 16)] = acc

        v0 = jnp.where((v0 == ms) & (iv[0] == mi), ninf, v0)
        v1 = jnp.where((v1 == ms) & (iv[1] == mi), ninf, v1)
        v2 = jnp.where((v2 == ms) & (iv[2] == mi), ninf, v2)
        v3 = jnp.where((v3 == ms) & (iv[3] == mi), ninf, v3)
        return (v0, v1, v2, v3, acc)

    lax.fori_loop(0, _K, kb, (*vv, jnp.zeros((16,), jnp.int32)))


def _sc_body(x_hbm, out_hbm, row_v, hist, sfx, cand, defb, outv):
    wid = lax.axis_index("s") * _NC + lax.axis_index("c")
    lane = _lane()
    ones = jnp.ones((16,), jnp.int32)

    def row_body(j, carry):
        r = wid * _RPW + j
        pltpu.sync_copy(
            x_hbm.at[pl.ds(pl.multiple_of(r * _COLS, _COLS), _COLS)], row_v)

        _clear_hist(hist)

        @plsc.parallel_loop(0, _NV, unroll=8)
        def _(i):
            v = row_v[pl.ds(pl.multiple_of(i * 16, 16), 16)]
            s = _skey(v)
            addr = ((lax.shift_right_logical(s, 20) & 0xFF0) ^ 0x800) + lane
            plsc.addupdate_scatter(hist, [addr], ones)

        b1, _ = _scan_bins(hist, sfx, jnp.int32(_K))
        sbound = (b1 ^ 0x80) << 24

        z16 = jnp.zeros((16,), jnp.int32)

        @plsc.parallel_loop(0, _NV, unroll=8, carry=(z16, z16))
        def cres(i, carry):
            off, base = carry
            s = _skey(row_v[pl.ds(pl.multiple_of(i * 16, 16), 16)])
            m = s >= sbound
            mi = m.astype(jnp.int32)
            plsc.store_scatter(cand, [plsc.cumsum(mi) - mi + off],
                               base + lane, mask=m)
            return (off + plsc.all_reduce_population_count(m), base + 16)

        cand_n = jnp.max(cres[0])

        def_n, cand_n = _filter(row_v, cand, defb, cand_n, jnp.int32(0),
                                b1, 24, True)
        for sh in (16, 8, 0):
            def_n, cand_n = _refine(row_v, hist, sfx, cand, defb,
                                    def_n, cand_n, sh)

        need_t = _K - def_n

        def abody(i, c):
            lm = (i * 16 + lane) < need_t
            ci = cand[pl.ds(pl.multiple_of(i * 16, 16), 16)] & 0x7FFF
            plsc.store_scatter(defb, [def_n + i * 16 + lane], ci, mask=lm)
            return c

        lax.fori_loop(0, (need_t + 15) // 16, abody, 0)

        _final_sort(row_v, defb, outv)
        pltpu.sync_copy(outv,
                        out_hbm.at[pl.ds(pl.multiple_of(r * _K, _K), _K)])
        return carry

    lax.fori_loop(0, _RPW, row_body, 0)


@functools.cache
def _sc_kernel():
    return pl.kernel(
        _sc_body,
        out_type=jax.ShapeDtypeStruct((_ROWS * _K,), jnp.int32),
        mesh=plsc.VectorSubcoreMesh(core_axis_name="c", subcore_axis_name="s",
                                    num_cores=_NC, num_subcores=_NS),
        scratch_types=[
            pltpu.VMEM((_COLS,), jnp.float32),
            pltpu.VMEM((4096,), jnp.int32),
            pltpu.VMEM((256,), jnp.int32),
            pltpu.VMEM((_CAND,), jnp.int32),
            pltpu.VMEM((96,), jnp.int32),
            pltpu.VMEM((_K,), jnp.int32),
        ],
        compiler_params=pltpu.CompilerParams(needs_layout_passes=False),
    )


def kernel(x):
    return _sc_kernel()(x.reshape(-1)).reshape(_ROWS, _K)


# speedup vs baseline: 10.3844x; 1.1891x over previous
import functools

import jax
import jax.numpy as jnp
from jax import lax
from jax.experimental import pallas as pl
from jax.experimental.pallas import tpu as pltpu
from jax.experimental.pallas import tpu_sc as plsc

_K = 64
_ROWS = 128
_COLS = 32768
_NC = 2
_NS = 16
_NW = _NC * _NS
_RPW = _ROWS // _NW
_NV = _COLS // 16
_CAND = _COLS + 16

_BIG = 2**30


def _lane():
    return lax.broadcasted_iota(jnp.int32, (16,), 0)


def _skey(v):
    b = lax.bitcast_convert_type(v, jnp.int32)
    return b ^ ((b >> 31) & jnp.int32(0x7FFFFFFF))


def _clear_hist(hist):
    z = jnp.zeros((16,), jnp.int32)

    @plsc.parallel_loop(0, 256, unroll=8)
    def _(i):
        hist[pl.ds(pl.multiple_of(i * 16, 16), 16)] = z


def _scan_bins(hist, sfx, need):
    lane = _lane()

    def chunk(t, r):
        c = 15 - t
        idx0 = c * 256 + lane * 16

        @plsc.parallel_loop(0, 16, unroll=4, carry=jnp.zeros((16,), jnp.int32))
        def acc(l, a):
            return a + plsc.load_gather(hist, [idx0 + l])
        cs = plsc.cumsum(lax.rev(acc, (0,)))
        sfx[pl.ds(pl.multiple_of(c * 16, 16), 16)] = lax.rev(cs, (0,)) + r
        return r + jnp.max(cs)

    lax.fori_loop(0, 16, chunk, jnp.int32(0))

    def cnt(c, acc):
        s = sfx[pl.ds(pl.multiple_of(c * 16, 16), 16)]
        return acc + (s >= need).astype(jnp.int32)

    accv = lax.fori_loop(0, 16, cnt, jnp.zeros((16,), jnp.int32))
    bbin = jnp.sum(accv) - 1
    g = plsc.load_gather(sfx, [jnp.broadcast_to(jnp.minimum(bbin + 1, 255), (16,))])
    above = jnp.where(bbin >= 255, jnp.int32(0), jnp.max(g))
    return bbin, above


def _filter(row_v, cand, defb, cand_n, def_n, bbin, sh, flip):
    lane = _lane()

    def fbody(i, carry):
        doff, coff = carry
        lm = (i * 16 + lane) < cand_n
        ci = cand[pl.ds(pl.multiple_of(i * 16, 16), 16)] & 0x7FFF
        s = _skey(plsc.load_gather(row_v, [ci]))
        byte = lax.shift_right_logical(s, sh) & 0xFF
        if flip:
            byte = byte ^ 0x80
        dm = (byte > bbin) & lm
        bm = (byte == bbin) & lm
        dmi = dm.astype(jnp.int32)
        bmi = bm.astype(jnp.int32)
        plsc.store_scatter(defb, [plsc.cumsum(dmi) - dmi + doff], ci, mask=dm)
        plsc.store_scatter(cand, [plsc.cumsum(bmi) - bmi + coff], ci, mask=bm)
        return (doff + plsc.all_reduce_population_count(dm),
                coff + plsc.all_reduce_population_count(bm))

    doff, coff = lax.fori_loop(
        0, (cand_n + 15) // 16, fbody,
        (jnp.broadcast_to(def_n, (16,)), jnp.zeros((16,), jnp.int32)))
    return jnp.max(doff), jnp.max(coff)


def _refine(row_v, hist, sfx, cand, defb, def_n, cand_n, sh):
    lane = _lane()
    ones = jnp.ones((16,), jnp.int32)

    def do(args):
        def_n, cand_n = args
        _clear_hist(hist)

        def hb(i, c):
            lm = (i * 16 + lane) < cand_n
            ci = cand[pl.ds(pl.multiple_of(i * 16, 16), 16)] & 0x7FFF
            s = _skey(plsc.load_gather(row_v, [ci]))
            byte = lax.shift_right_logical(s, sh) & 0xFF
            plsc.addupdate_scatter(hist, [(byte << 4) + lane], ones, mask=lm)
            return c

        lax.fori_loop(0, (cand_n + 15) // 16, hb, 0)
        bbin, _ = _scan_bins(hist, sfx, _K - def_n)
        return _filter(row_v, cand, defb, cand_n, def_n, bbin, sh, False)

    return lax.cond(cand_n > _K - def_n, do, lambda a: a, (def_n, cand_n))


def _final_sort(row_v, defb, outv):
    lane = _lane()
    iv = [defb[pl.ds(16 * j, 16)] for j in range(4)]
    vv = [plsc.load_gather(row_v, [iv[j] & 0x7FFF]) for j in range(4)]
    ninf = jnp.float32(-jnp.inf)

    def kb(k, carry):
        v0, v1, v2, v3, acc = carry
        ms = jnp.max(jnp.maximum(jnp.maximum(v0, v1), jnp.maximum(v2, v3)))
        c0 = jnp.where(v0 == ms, iv[0], _BIG)
        c1 = jnp.where(v1 == ms, iv[1], _BIG)
        c2 = jnp.where(v2 == ms, iv[2], _BIG)
        c3 = jnp.where(v3 == ms, iv[3], _BIG)
        mi = jnp.min(jnp.minimum(jnp.minimum(c0, c1), jnp.minimum(c2, c3)))
        acc = jnp.where(lane == (k & 15), mi, acc)

        @pl.when((k & 15) == 15)
        def _():
            outv[pl.ds(pl.multiple_of(k - 15, 16), 16)] = acc

        v0 = jnp.where((v0 == ms) & (iv[0] == mi), ninf, v0)
        v1 = jnp.where((v1 == ms) & (iv[1] == mi), ninf, v1)
        v2 = jnp.where((v2 == ms) & (iv[2] == mi), ninf, v2)
        v3 = jnp.where((v3 == ms) & (iv[3] == mi), ninf, v3)
        return (v0, v1, v2, v3, acc)

    lax.fori_loop(0, _K, kb, (*vv, jnp.zeros((16,), jnp.int32)))


def _sc_body(x_hbm, out_hbm, row_v, hist, sfx, cand, defb, outv):
    wid = lax.axis_index("s") * _NC + lax.axis_index("c")
    lane = _lane()
    ones = jnp.ones((16,), jnp.int32)

    def row_body(j, carry):
        r = wid * _RPW + j
        pltpu.sync_copy(x_hbm.at[r], row_v)

        _clear_hist(hist)

        @plsc.parallel_loop(0, _NV, unroll=8)
        def _(i):
            v = row_v[pl.ds(pl.multiple_of(i * 16, 16), 16)]
            s = _skey(v)
            addr = ((lax.shift_right_logical(s, 20) & 0xFF0) ^ 0x800) + lane
            plsc.addupdate_scatter(hist, [addr], ones)

        b1, _ = _scan_bins(hist, sfx, jnp.int32(_K))
        sbound = (b1 ^ 0x80) << 24

        z16 = jnp.zeros((16,), jnp.int32)

        @plsc.parallel_loop(0, _NV, unroll=8, carry=(z16, z16))
        def cres(i, carry):
            off, base = carry
            s = _skey(row_v[pl.ds(pl.multiple_of(i * 16, 16), 16)])
            m = s >= sbound
            mi = m.astype(jnp.int32)
            plsc.store_scatter(cand, [plsc.cumsum(mi) - mi + off],
                               base + lane, mask=m)
            return (off + plsc.all_reduce_population_count(m), base + 16)

        cand_n = jnp.max(cres[0])

        def_n, cand_n = _filter(row_v, cand, defb, cand_n, jnp.int32(0),
                                b1, 24, True)
        for sh in (16, 8, 0):
            def_n, cand_n = _refine(row_v, hist, sfx, cand, defb,
                                    def_n, cand_n, sh)

        need_t = _K - def_n

        def abody(i, c):
            lm = (i * 16 + lane) < need_t
            ci = cand[pl.ds(pl.multiple_of(i * 16, 16), 16)] & 0x7FFF
            plsc.store_scatter(defb, [def_n + i * 16 + lane], ci, mask=lm)
            return c

        lax.fori_loop(0, (need_t + 15) // 16, abody, 0)

        _final_sort(row_v, defb, outv)
        pltpu.sync_copy(outv,
                        out_hbm.at[pl.ds(pl.multiple_of(r * _K, _K), _K)])
        return carry

    lax.fori_loop(0, _RPW, row_body, 0)


@functools.cache
def _sc_kernel():
    return pl.kernel(
        _sc_body,
        out_type=jax.ShapeDtypeStruct((_ROWS * _K,), jnp.int32),
        mesh=plsc.VectorSubcoreMesh(core_axis_name="c", subcore_axis_name="s",
                                    num_cores=_NC, num_subcores=_NS),
        scratch_types=[
            pltpu.VMEM((_COLS,), jnp.float32),
            pltpu.VMEM((4096,), jnp.int32),
            pltpu.VMEM((256,), jnp.int32),
            pltpu.VMEM((_CAND,), jnp.int32),
            pltpu.VMEM((96,), jnp.int32),
            pltpu.VMEM((_K,), jnp.int32),
        ],
        compiler_params=pltpu.CompilerParams(needs_layout_passes=False),
    )


def kernel(x):
    return _sc_kernel()(x).reshape(_ROWS, _K)


# speedup vs baseline: 11.0680x; 1.0658x over previous
import functools

import jax
import jax.numpy as jnp
from jax import lax
from jax.experimental import pallas as pl
from jax.experimental.pallas import tpu as pltpu
from jax.experimental.pallas import tpu_sc as plsc

_K = 64
_ROWS = 128
_COLS = 32768
_NC = 2
_NS = 16
_NW = _NC * _NS
_RPW = _ROWS // _NW
_NV = _COLS // 16
_CAND = _COLS + 16

_BIG = 2**30


def _lane():
    return lax.broadcasted_iota(jnp.int32, (16,), 0)


def _skey(v):
    b = lax.bitcast_convert_type(v, jnp.int32)
    return b ^ ((b >> 31) & jnp.int32(0x7FFFFFFF))


def _clear_hist(hist):
    z = jnp.zeros((16,), jnp.int32)

    @plsc.parallel_loop(0, 256, unroll=8)
    def _(i):
        hist[pl.ds(pl.multiple_of(i * 16, 16), 16)] = z


def _scan_bins(hist, sfx, need):
    lane = _lane()

    def chunk(t, r):
        c = 15 - t
        idx0 = c * 256 + lane * 16

        @plsc.parallel_loop(0, 16, unroll=4, carry=jnp.zeros((16,), jnp.int32))
        def acc(l, a):
            return a + plsc.load_gather(hist, [idx0 + l])
        cs = plsc.cumsum(lax.rev(acc, (0,)))
        sfx[pl.ds(pl.multiple_of(c * 16, 16), 16)] = lax.rev(cs, (0,)) + r
        return r + jnp.max(cs)

    lax.fori_loop(0, 16, chunk, jnp.int32(0))

    def cnt(c, acc):
        s = sfx[pl.ds(pl.multiple_of(c * 16, 16), 16)]
        return acc + (s >= need).astype(jnp.int32)

    accv = lax.fori_loop(0, 16, cnt, jnp.zeros((16,), jnp.int32))
    bbin = jnp.sum(accv) - 1
    g = plsc.load_gather(sfx, [jnp.broadcast_to(jnp.minimum(bbin + 1, 255), (16,))])
    above = jnp.where(bbin >= 255, jnp.int32(0), jnp.max(g))
    return bbin, above


def _filter(row_v, cand, defb, cand_n, def_n, bbin, sh, flip):
    lane = _lane()

    def fbody(i, carry):
        doff, coff = carry
        lm = (i * 16 + lane) < cand_n
        ci = cand[pl.ds(pl.multiple_of(i * 16, 16), 16)] & 0x7FFF
        s = _skey(plsc.load_gather(row_v, [ci]))
        byte = lax.shift_right_logical(s, sh) & 0xFF
        if flip:
            byte = byte ^ 0x80
        dm = (byte > bbin) & lm
        bm = (byte == bbin) & lm
        dmi = dm.astype(jnp.int32)
        bmi = bm.astype(jnp.int32)
        plsc.store_scatter(defb, [plsc.cumsum(dmi) - dmi + doff], ci, mask=dm)
        plsc.store_scatter(cand, [plsc.cumsum(bmi) - bmi + coff], ci, mask=bm)
        return (doff + plsc.all_reduce_population_count(dm),
                coff + plsc.all_reduce_population_count(bm))

    doff, coff = lax.fori_loop(
        0, (cand_n + 15) // 16, fbody,
        (jnp.broadcast_to(def_n, (16,)), jnp.zeros((16,), jnp.int32)))
    return jnp.max(doff), jnp.max(coff)


def _refine(row_v, hist, sfx, cand, defb, def_n, cand_n, sh):
    lane = _lane()
    ones = jnp.ones((16,), jnp.int32)

    def do(args):
        def_n, cand_n = args
        _clear_hist(hist)

        def hb(i, c):
            lm = (i * 16 + lane) < cand_n
            ci = cand[pl.ds(pl.multiple_of(i * 16, 16), 16)] & 0x7FFF
            s = _skey(plsc.load_gather(row_v, [ci]))
            byte = lax.shift_right_logical(s, sh) & 0xFF
            plsc.addupdate_scatter(hist, [(byte << 4) + lane], ones, mask=lm)
            return c

        lax.fori_loop(0, (cand_n + 15) // 16, hb, 0)
        bbin, _ = _scan_bins(hist, sfx, _K - def_n)
        return _filter(row_v, cand, defb, cand_n, def_n, bbin, sh, False)

    return lax.cond(cand_n > _K - def_n, do, lambda a: a, (def_n, cand_n))


def _final_sort(row_v, defb, outv):
    lane = _lane()
    iv = [defb[pl.ds(16 * j, 16)] for j in range(4)]
    vv = [plsc.load_gather(row_v, [iv[j] & 0x7FFF]) for j in range(4)]
    ninf = jnp.float32(-jnp.inf)

    def kb(k, carry):
        v0, v1, v2, v3, acc = carry
        ms = jnp.max(jnp.maximum(jnp.maximum(v0, v1), jnp.maximum(v2, v3)))
        c0 = jnp.where(v0 == ms, iv[0], _BIG)
        c1 = jnp.where(v1 == ms, iv[1], _BIG)
        c2 = jnp.where(v2 == ms, iv[2], _BIG)
        c3 = jnp.where(v3 == ms, iv[3], _BIG)
        mi = jnp.min(jnp.minimum(jnp.minimum(c0, c1), jnp.minimum(c2, c3)))
        acc = jnp.where(lane == (k & 15), mi, acc)

        @pl.when((k & 15) == 15)
        def _():
            outv[pl.ds(pl.multiple_of(k - 15, 16), 16)] = acc

        v0 = jnp.where((v0 == ms) & (iv[0] == mi), ninf, v0)
        v1 = jnp.where((v1 == ms) & (iv[1] == mi), ninf, v1)
        v2 = jnp.where((v2 == ms) & (iv[2] == mi), ninf, v2)
        v3 = jnp.where((v3 == ms) & (iv[3] == mi), ninf, v3)
        return (v0, v1, v2, v3, acc)

    lax.fori_loop(0, _K, kb, (*vv, jnp.zeros((16,), jnp.int32)))


def _sc_body(x_hbm, out_hbm, rows_v, hist, sfx, cand, defb, outv, sem):
    wid = lax.axis_index("s") * _NC + lax.axis_index("c")
    lane = _lane()
    ones = jnp.ones((16,), jnp.int32)

    r0 = wid * _RPW
    pltpu.async_copy(x_hbm.at[r0], rows_v.at[pl.ds(0, _COLS)], sem)

    def row_body(j, carry):
        r = r0 + j
        pbase = pl.multiple_of((j & 1) * _COLS, _COLS)
        row_v = rows_v.at[pl.ds(pbase, _COLS)]
        pltpu.make_async_copy(x_hbm.at[r], row_v, sem).wait()

        @pl.when(j < _RPW - 1)
        def _():
            nbase = pl.multiple_of(((j + 1) & 1) * _COLS, _COLS)
            pltpu.async_copy(x_hbm.at[r + 1],
                             rows_v.at[pl.ds(nbase, _COLS)], sem)

        _clear_hist(hist)

        @plsc.parallel_loop(0, _NV, unroll=8)
        def _(i):
            v = row_v[pl.ds(pl.multiple_of(i * 16, 16), 16)]
            s = _skey(v)
            addr = ((lax.shift_right_logical(s, 20) & 0xFF0) ^ 0x800) + lane
            plsc.addupdate_scatter(hist, [addr], ones)

        b1, _ = _scan_bins(hist, sfx, jnp.int32(_K))
        sbound = (b1 ^ 0x80) << 24

        z16 = jnp.zeros((16,), jnp.int32)

        @plsc.parallel_loop(0, _NV, unroll=8, carry=(z16, z16))
        def cres(i, carry):
            off, base = carry
            s = _skey(row_v[pl.ds(pl.multiple_of(i * 16, 16), 16)])
            m = s >= sbound
            mi = m.astype(jnp.int32)
            plsc.store_scatter(cand, [plsc.cumsum(mi) - mi + off],
                               base + lane, mask=m)
            return (off + plsc.all_reduce_population_count(m), base + 16)

        cand_n = jnp.max(cres[0])

        def_n, cand_n = _filter(row_v, cand, defb, cand_n, jnp.int32(0),
                                b1, 24, True)
        for sh in (16, 8, 0):
            def_n, cand_n = _refine(row_v, hist, sfx, cand, defb,
                                    def_n, cand_n, sh)

        need_t = _K - def_n

        def abody(i, c):
            lm = (i * 16 + lane) < need_t
            ci = cand[pl.ds(pl.multiple_of(i * 16, 16), 16)] & 0x7FFF
            plsc.store_scatter(defb, [def_n + i * 16 + lane], ci, mask=lm)
            return c

        lax.fori_loop(0, (need_t + 15) // 16, abody, 0)

        _final_sort(row_v, defb, outv)
        pltpu.sync_copy(outv,
                        out_hbm.at[pl.ds(pl.multiple_of(r * _K, _K), _K)])
        return carry

    lax.fori_loop(0, _RPW, row_body, 0)


@functools.cache
def _sc_kernel():
    return pl.kernel(
        _sc_body,
        out_type=jax.ShapeDtypeStruct((_ROWS * _K,), jnp.int32),
        mesh=plsc.VectorSubcoreMesh(core_axis_name="c", subcore_axis_name="s",
                                    num_cores=_NC, num_subcores=_NS),
        scratch_types=[
            pltpu.VMEM((2 * _COLS,), jnp.float32),
            pltpu.VMEM((4096,), jnp.int32),
            pltpu.VMEM((256,), jnp.int32),
            pltpu.VMEM((_CAND,), jnp.int32),
            pltpu.VMEM((96,), jnp.int32),
            pltpu.VMEM((_K,), jnp.int32),
            pltpu.SemaphoreType.DMA,
        ],
        compiler_params=pltpu.CompilerParams(needs_layout_passes=False),
    )


def kernel(x):
    return _sc_kernel()(x).reshape(_ROWS, _K)


# speedup vs baseline: 11.8881x; 1.0741x over previous
import functools

import jax
import jax.numpy as jnp
from jax import lax
from jax.experimental import pallas as pl
from jax.experimental.pallas import tpu as pltpu
from jax.experimental.pallas import tpu_sc as plsc

_K = 64
_ROWS = 128
_COLS = 32768
_NC = 2
_NS = 16
_NW = _NC * _NS
_RPW = _ROWS // _NW
_NV = _COLS // 16
_CAND = _COLS + 16

_BIG = 2**30


def _lane():
    return lax.broadcasted_iota(jnp.int32, (16,), 0)


def _skey(v):
    b = lax.bitcast_convert_type(v, jnp.int32)
    return b ^ ((b >> 31) & jnp.int32(0x7FFFFFFF))


def _clear_hist(hist):
    z = jnp.zeros((16,), jnp.int32)

    @plsc.parallel_loop(0, 256, unroll=8)
    def _(i):
        hist[pl.ds(pl.multiple_of(i * 16, 16), 16)] = z


def _scan_bins(hist, sfx, rbuf, need):
    lane = _lane()

    @plsc.parallel_loop(0, 16, unroll=2)
    def _(c):
        idx0 = c * 256 + lane * 16
        gs = [plsc.load_gather(hist, [idx0 + l]) for l in range(16)]
        while len(gs) > 1:
            gs = [a + b for a, b in zip(gs[::2], gs[1::2])]
        rcs = plsc.cumsum(lax.rev(gs[0], (0,)))
        sfx[pl.ds(pl.multiple_of(c * 16, 16), 16)] = lax.rev(rcs, (0,))

    totals = plsc.load_gather(sfx, [lane * 16])
    inc = lax.rev(plsc.cumsum(lax.rev(totals, (0,))), (0,))
    rbuf[pl.ds(0, 16)] = inc - totals

    @plsc.parallel_loop(0, 16, unroll=2)
    def _(c):
        rsp = plsc.load_gather(rbuf, [jnp.broadcast_to(c, (16,))])
        off = pl.multiple_of(c * 16, 16)
        sfx[pl.ds(off, 16)] = sfx[pl.ds(off, 16)] + rsp

    @plsc.parallel_loop(0, 16, unroll=4, carry=jnp.zeros((16,), jnp.int32))
    def accv(c, a):
        s = sfx[pl.ds(pl.multiple_of(c * 16, 16), 16)]
        return a + (s >= need).astype(jnp.int32)

    bbin = jnp.sum(accv) - 1
    g = plsc.load_gather(sfx, [jnp.broadcast_to(jnp.minimum(bbin + 1, 255), (16,))])
    above = jnp.where(bbin >= 255, jnp.int32(0), jnp.max(g))
    return bbin, above


def _filter(row_v, cand, defb, cand_n, def_n, bbin, sh, flip):
    lane = _lane()

    def fbody(i, carry):
        doff, coff = carry
        lm = (i * 16 + lane) < cand_n
        ci = cand[pl.ds(pl.multiple_of(i * 16, 16), 16)] & 0x7FFF
        s = _skey(plsc.load_gather(row_v, [ci]))
        byte = lax.shift_right_logical(s, sh) & 0xFF
        if flip:
            byte = byte ^ 0x80
        dm = (byte > bbin) & lm
        bm = (byte == bbin) & lm
        dmi = dm.astype(jnp.int32)
        bmi = bm.astype(jnp.int32)
        plsc.store_scatter(defb, [plsc.cumsum(dmi) - dmi + doff], ci, mask=dm)
        plsc.store_scatter(cand, [plsc.cumsum(bmi) - bmi + coff], ci, mask=bm)
        return (doff + plsc.all_reduce_population_count(dm),
                coff + plsc.all_reduce_population_count(bm))

    doff, coff = lax.fori_loop(
        0, (cand_n + 15) // 16, fbody,
        (jnp.broadcast_to(def_n, (16,)), jnp.zeros((16,), jnp.int32)))
    return jnp.max(doff), jnp.max(coff)


def _refine(row_v, hist, sfx, rbuf, cand, defb, def_n, cand_n, sh):
    lane = _lane()
    ones = jnp.ones((16,), jnp.int32)

    def do(args):
        def_n, cand_n = args
        _clear_hist(hist)

        @plsc.parallel_loop(0, (cand_n + 15) // 16, unroll=2)
        def _(i):
            lm = (i * 16 + lane) < cand_n
            ci = cand[pl.ds(pl.multiple_of(i * 16, 16), 16)] & 0x7FFF
            s = _skey(plsc.load_gather(row_v, [ci]))
            byte = lax.shift_right_logical(s, sh) & 0xFF
            plsc.addupdate_scatter(hist, [(byte << 4) + lane], ones, mask=lm)

        bbin, _ = _scan_bins(hist, sfx, rbuf, _K - def_n)
        return _filter(row_v, cand, defb, cand_n, def_n, bbin, sh, False)

    return lax.cond(cand_n > _K - def_n, do, lambda a: a, (def_n, cand_n))


def _final_sort(row_v, defb, outv):
    lane = _lane()
    iv = [defb[pl.ds(16 * j, 16)] for j in range(4)]
    vv = [plsc.load_gather(row_v, [iv[j] & 0x7FFF]) for j in range(4)]
    ninf = jnp.float32(-jnp.inf)

    def kb(k, carry):
        v0, v1, v2, v3, acc = carry
        ms = jnp.max(jnp.maximum(jnp.maximum(v0, v1), jnp.maximum(v2, v3)))
        c0 = jnp.where(v0 == ms, iv[0], _BIG)
        c1 = jnp.where(v1 == ms, iv[1], _BIG)
        c2 = jnp.where(v2 == ms, iv[2], _BIG)
        c3 = jnp.where(v3 == ms, iv[3], _BIG)
        mi = jnp.min(jnp.minimum(jnp.minimum(c0, c1), jnp.minimum(c2, c3)))
        acc = jnp.where(lane == (k & 15), mi, acc)

        @pl.when((k & 15) == 15)
        def _():
            outv[pl.ds(pl.multiple_of(k - 15, 16), 16)] = acc

        v0 = jnp.where((v0 == ms) & (iv[0] == mi), ninf, v0)
        v1 = jnp.where((v1 == ms) & (iv[1] == mi), ninf, v1)
        v2 = jnp.where((v2 == ms) & (iv[2] == mi), ninf, v2)
        v3 = jnp.where((v3 == ms) & (iv[3] == mi), ninf, v3)
        return (v0, v1, v2, v3, acc)

    lax.fori_loop(0, _K, kb, (*vv, jnp.zeros((16,), jnp.int32)))


def _sc_body(x_hbm, out_hbm, rows_v, hist, sfx, rbuf, cand, defb, outv, sem):
    wid = lax.axis_index("s") * _NC + lax.axis_index("c")
    lane = _lane()
    ones = jnp.ones((16,), jnp.int32)

    r0 = wid * _RPW
    pltpu.async_copy(x_hbm.at[r0], rows_v.at[pl.ds(0, _COLS)], sem)

    def row_body(j, carry):
        r = r0 + j
        pbase = pl.multiple_of((j & 1) * _COLS, _COLS)
        row_v = rows_v.at[pl.ds(pbase, _COLS)]
        pltpu.make_async_copy(x_hbm.at[r], row_v, sem).wait()

        @pl.when(j < _RPW - 1)
        def _():
            nbase = pl.multiple_of(((j + 1) & 1) * _COLS, _COLS)
            pltpu.async_copy(x_hbm.at[r + 1],
                             rows_v.at[pl.ds(nbase, _COLS)], sem)

        _clear_hist(hist)

        @plsc.parallel_loop(0, _NV, unroll=8)
        def _(i):
            v = row_v[pl.ds(pl.multiple_of(i * 16, 16), 16)]
            s = _skey(v)
            addr = ((lax.shift_right_logical(s, 20) & 0xFF0) ^ 0x800) + lane
            plsc.addupdate_scatter(hist, [addr], ones)

        b1, _ = _scan_bins(hist, sfx, rbuf, jnp.int32(_K))
        sbound = (b1 ^ 0x80) << 24

        z16 = jnp.zeros((16,), jnp.int32)

        @plsc.parallel_loop(0, _NV, unroll=8, carry=(z16, z16))
        def cres(i, carry):
            off, base = carry
            s = _skey(row_v[pl.ds(pl.multiple_of(i * 16, 16), 16)])
            m = s >= sbound
            mi = m.astype(jnp.int32)
            plsc.store_scatter(cand, [plsc.cumsum(mi) - mi + off],
                               base + lane, mask=m)
            return (off + plsc.all_reduce_population_count(m), base + 16)

        cand_n = jnp.max(cres[0])

        def_n, cand_n = _filter(row_v, cand, defb, cand_n, jnp.int32(0),
                                b1, 24, True)
        for sh in (16, 8, 0):
            def_n, cand_n = _refine(row_v, hist, sfx, rbuf, cand, defb,
                                    def_n, cand_n, sh)

        need_t = _K - def_n

        def abody(i, c):
            lm = (i * 16 + lane) < need_t
            ci = cand[pl.ds(pl.multiple_of(i * 16, 16), 16)] & 0x7FFF
            plsc.store_scatter(defb, [def_n + i * 16 + lane], ci, mask=lm)
            return c

        lax.fori_loop(0, (need_t + 15) // 16, abody, 0)

        _final_sort(row_v, defb, outv)
        pltpu.sync_copy(outv,
                        out_hbm.at[pl.ds(pl.multiple_of(r * _K, _K), _K)])
        return carry

    lax.fori_loop(0, _RPW, row_body, 0)


@functools.cache
def _sc_kernel():
    return pl.kernel(
        _sc_body,
        out_type=jax.ShapeDtypeStruct((_ROWS * _K,), jnp.int32),
        mesh=plsc.VectorSubcoreMesh(core_axis_name="c", subcore_axis_name="s",
                                    num_cores=_NC, num_subcores=_NS),
        scratch_types=[
            pltpu.VMEM((2 * _COLS,), jnp.float32),
            pltpu.VMEM((4096,), jnp.int32),
            pltpu.VMEM((256,), jnp.int32),
            pltpu.VMEM((16,), jnp.int32),
            pltpu.VMEM((_CAND,), jnp.int32),
            pltpu.VMEM((96,), jnp.int32),
            pltpu.VMEM((_K,), jnp.int32),
            pltpu.SemaphoreType.DMA,
        ],
        compiler_params=pltpu.CompilerParams(needs_layout_passes=False),
    )


def kernel(x):
    return _sc_kernel()(x).reshape(_ROWS, _K)


# speedup vs baseline: 12.1092x; 1.0186x over previous
import functools

import jax
import jax.numpy as jnp
from jax import lax
from jax.experimental import pallas as pl
from jax.experimental.pallas import tpu as pltpu
from jax.experimental.pallas import tpu_sc as plsc

_K = 64
_ROWS = 128
_COLS = 32768
_NC = 2
_NS = 16
_NW = _NC * _NS
_RPW = _ROWS // _NW
_NV = _COLS // 16
_CAND = _COLS + 16

_BIG = 2**30


def _lane():
    return lax.broadcasted_iota(jnp.int32, (16,), 0)


def _skey(v):
    b = lax.bitcast_convert_type(v, jnp.int32)
    return b ^ ((b >> 31) & jnp.int32(0x7FFFFFFF))


def _clear_hist(hist):
    z = jnp.zeros((16,), jnp.int32)

    @plsc.parallel_loop(0, 256, unroll=8)
    def _(i):
        hist[pl.ds(pl.multiple_of(i * 16, 16), 16)] = z


def _scan_bins(hist, sfx, rbuf, need):
    lane = _lane()

    @plsc.parallel_loop(0, 16, unroll=2)
    def _(c):
        idx0 = c * 256 + lane * 16
        gs = [plsc.load_gather(hist, [idx0 + l]) for l in range(16)]
        while len(gs) > 1:
            gs = [a + b for a, b in zip(gs[::2], gs[1::2])]
        rcs = plsc.cumsum(lax.rev(gs[0], (0,)))
        sfx[pl.ds(pl.multiple_of(c * 16, 16), 16)] = lax.rev(rcs, (0,))

    totals = plsc.load_gather(sfx, [lane * 16])
    inc = lax.rev(plsc.cumsum(lax.rev(totals, (0,))), (0,))
    rbuf[pl.ds(0, 16)] = inc - totals

    @plsc.parallel_loop(0, 16, unroll=2)
    def _(c):
        rsp = plsc.load_gather(rbuf, [jnp.broadcast_to(c, (16,))])
        off = pl.multiple_of(c * 16, 16)
        sfx[pl.ds(off, 16)] = sfx[pl.ds(off, 16)] + rsp

    @plsc.parallel_loop(0, 16, unroll=4, carry=jnp.zeros((16,), jnp.int32))
    def accv(c, a):
        s = sfx[pl.ds(pl.multiple_of(c * 16, 16), 16)]
        return a + (s >= need).astype(jnp.int32)

    bbin = jnp.sum(accv) - 1
    g = plsc.load_gather(sfx, [jnp.broadcast_to(jnp.minimum(bbin + 1, 255), (16,))])
    above = jnp.where(bbin >= 255, jnp.int32(0), jnp.max(g))
    return bbin, above


def _filter(row_v, cand, defb, cand_n, def_n, bbin, sh, flip):
    lane = _lane()

    def fbody(i, carry):
        doff, coff = carry
        lm = (i * 16 + lane) < cand_n
        ci = cand[pl.ds(pl.multiple_of(i * 16, 16), 16)] & 0x7FFF
        s = _skey(plsc.load_gather(row_v, [ci]))
        byte = lax.shift_right_logical(s, sh) & 0xFF
        if flip:
            byte = byte ^ 0x80
        dm = (byte > bbin) & lm
        bm = (byte == bbin) & lm
        dmi = dm.astype(jnp.int32)
        bmi = bm.astype(jnp.int32)
        plsc.store_scatter(defb, [plsc.cumsum(dmi) - dmi + doff], ci, mask=dm)
        plsc.store_scatter(cand, [plsc.cumsum(bmi) - bmi + coff], ci, mask=bm)
        return (doff + plsc.all_reduce_population_count(dm),
                coff + plsc.all_reduce_population_count(bm))

    doff, coff = lax.fori_loop(
        0, (cand_n + 15) // 16, fbody,
        (jnp.broadcast_to(def_n, (16,)), jnp.zeros((16,), jnp.int32)))
    return jnp.max(doff), jnp.max(coff)


def _refine(row_v, hist, sfx, rbuf, cand, defb, def_n, cand_n, sh):
    lane = _lane()
    ones = jnp.ones((16,), jnp.int32)

    def do(args):
        def_n, cand_n = args
        _clear_hist(hist)

        @plsc.parallel_loop(0, (cand_n + 15) // 16, unroll=2)
        def _(i):
            lm = (i * 16 + lane) < cand_n
            ci = cand[pl.ds(pl.multiple_of(i * 16, 16), 16)] & 0x7FFF
            s = _skey(plsc.load_gather(row_v, [ci]))
            byte = lax.shift_right_logical(s, sh) & 0xFF
            plsc.addupdate_scatter(hist, [(byte << 4) + lane], ones, mask=lm)

        bbin, _ = _scan_bins(hist, sfx, rbuf, _K - def_n)
        return _filter(row_v, cand, defb, cand_n, def_n, bbin, sh, False)

    return lax.cond(cand_n > _K - def_n, do, lambda a: a, (def_n, cand_n))


def _final_sort(row_v, defb, outv):
    lane = _lane()
    iv = [defb[pl.ds(16 * j, 16)] for j in range(4)]
    sv = [_skey(plsc.load_gather(row_v, [iv[j] & 0x7FFF])) for j in range(4)]
    ranks = [jnp.zeros((16,), jnp.int32) for _ in range(4)]
    for r in range(16):
        ridx = (lane + r) & 15
        for j2 in range(4):
            s2 = sv[j2].at[ridx].get(mode="promise_in_bounds")
            i2 = iv[j2].at[ridx].get(mode="promise_in_bounds")
            for j in range(4):
                beats = (s2 > sv[j]) | ((s2 == sv[j]) & (i2 < iv[j]))
                ranks[j] = ranks[j] + beats.astype(jnp.int32)
    for j in range(4):
        plsc.store_scatter(outv, [ranks[j]], iv[j])


def _sc_body(x_hbm, out_hbm, rows_v, hist, sfx, rbuf, cand, defb, outv, sem):
    wid = lax.axis_index("s") * _NC + lax.axis_index("c")
    lane = _lane()
    ones = jnp.ones((16,), jnp.int32)

    r0 = wid * _RPW
    pltpu.async_copy(x_hbm.at[r0], rows_v.at[pl.ds(0, _COLS)], sem)

    def row_body(j, carry):
        r = r0 + j
        pbase = pl.multiple_of((j & 1) * _COLS, _COLS)
        row_v = rows_v.at[pl.ds(pbase, _COLS)]
        pltpu.make_async_copy(x_hbm.at[r], row_v, sem).wait()

        @pl.when(j < _RPW - 1)
        def _():
            nbase = pl.multiple_of(((j + 1) & 1) * _COLS, _COLS)
            pltpu.async_copy(x_hbm.at[r + 1],
                             rows_v.at[pl.ds(nbase, _COLS)], sem)

        _clear_hist(hist)

        @plsc.parallel_loop(0, _NV, unroll=8)
        def _(i):
            v = row_v[pl.ds(pl.multiple_of(i * 16, 16), 16)]
            s = _skey(v)
            addr = ((lax.shift_right_logical(s, 20) & 0xFF0) ^ 0x800) + lane
            plsc.addupdate_scatter(hist, [addr], ones)

        b1, _ = _scan_bins(hist, sfx, rbuf, jnp.int32(_K))
        sbound = (b1 ^ 0x80) << 24

        z16 = jnp.zeros((16,), jnp.int32)

        @plsc.parallel_loop(0, _NV, unroll=8, carry=(z16, z16))
        def cres(i, carry):
            off, base = carry
            s = _skey(row_v[pl.ds(pl.multiple_of(i * 16, 16), 16)])
            m = s >= sbound
            mi = m.astype(jnp.int32)
            plsc.store_scatter(cand, [plsc.cumsum(mi) - mi + off],
                               base + lane, mask=m)
            return (off + plsc.all_reduce_population_count(m), base + 16)

        cand_n = jnp.max(cres[0])

        def_n, cand_n = _filter(row_v, cand, defb, cand_n, jnp.int32(0),
                                b1, 24, True)
        for sh in (16, 8, 0):
            def_n, cand_n = _refine(row_v, hist, sfx, rbuf, cand, defb,
                                    def_n, cand_n, sh)

        need_t = _K - def_n

        def abody(i, c):
            lm = (i * 16 + lane) < need_t
            ci = cand[pl.ds(pl.multiple_of(i * 16, 16), 16)] & 0x7FFF
            plsc.store_scatter(defb, [def_n + i * 16 + lane], ci, mask=lm)
            return c

        lax.fori_loop(0, (need_t + 15) // 16, abody, 0)

        _final_sort(row_v, defb, outv)
        pltpu.sync_copy(outv,
                        out_hbm.at[pl.ds(pl.multiple_of(r * _K, _K), _K)])
        return carry

    lax.fori_loop(0, _RPW, row_body, 0)


@functools.cache
def _sc_kernel():
    return pl.kernel(
        _sc_body,
        out_type=jax.ShapeDtypeStruct((_ROWS * _K,), jnp.int32),
        mesh=plsc.VectorSubcoreMesh(core_axis_name="c", subcore_axis_name="s",
                                    num_cores=_NC, num_subcores=_NS),
        scratch_types=[
            pltpu.VMEM((2 * _COLS,), jnp.float32),
            pltpu.VMEM((4096,), jnp.int32),
            pltpu.VMEM((256,), jnp.int32),
            pltpu.VMEM((16,), jnp.int32),
            pltpu.VMEM((_CAND,), jnp.int32),
            pltpu.VMEM((96,), jnp.int32),
            pltpu.VMEM((_K,), jnp.int32),
            pltpu.SemaphoreType.DMA,
        ],
        compiler_params=pltpu.CompilerParams(needs_layout_passes=False),
    )


def kernel(x):
    return _sc_kernel()(x).reshape(_ROWS, _K)
